# Initial kernel scaffold; baseline (speedup 1.0000x reference)
import jax
import jax.numpy as jnp
from jax.experimental import pallas as pl


def kernel(inputs, W1, W2, Wr):
    raise NotImplementedError("write your pallas kernel here")


# speedup vs baseline: 3.6413x; 3.6413x over previous
import functools
import jax
import jax.numpy as jnp
from jax import lax
from jax.experimental import pallas as pl
from jax.experimental.pallas import tpu as pltpu
from jax.experimental.pallas import tpu_sc as plsc

_B, _S = 2, 2048
_D = 1024
_H = 2048
_E = 8
_N = _B * _S
_TT = 256
_TPE = _N // _E
_NI = _TPE // _TT

_NC, _NS = 2, 16
_NW = _NC * _NS
_RPW = 2 * _N // _NW
_CH = 64
_NCH = _RPW // _CH


def _router_body(x_ref, wr_ref, probs_ref, gidx_ref, gates_ref):
    x = x_ref[...]
    logits = lax.dot_general(x, wr_ref[...], (((1,), (1,)), ((), ())),
                             preferred_element_type=jnp.float32)
    p1 = jax.nn.softmax(logits, axis=-1)
    probs = jax.nn.softmax(p1, axis=-1)
    lane = lax.broadcasted_iota(jnp.int32, (_N, _E), 1)
    m1 = jnp.max(probs, axis=-1, keepdims=True)
    i1 = jnp.min(jnp.where(probs == m1, lane, _E), axis=-1, keepdims=True)
    p_wo = jnp.where(lane == i1, -1.0, probs)
    m2 = jnp.max(p_wo, axis=-1, keepdims=True)
    i2 = jnp.min(jnp.where(p_wo == m2, lane, _E), axis=-1, keepdims=True)
    row = lax.broadcasted_iota(jnp.int32, (_N, 1), 0)
    base8 = 8 * (row % _TPE)
    probs_ref[...] = probs
    gidx_ref[:, 0:1] = base8 + i1
    gidx_ref[:, 1:2] = base8 + i2
    gates_ref[:, 0:1] = m1
    gates_ref[:, 1:2] = m2


def _router_call(x, Wr):
    return pl.pallas_call(
        _router_body,
        in_specs=[
            pl.BlockSpec((_N, _D), lambda: (0, 0)),
            pl.BlockSpec((_E, _D), lambda: (0, 0)),
        ],
        out_specs=[
            pl.BlockSpec((_N, _E), lambda: (0, 0)),
            pl.BlockSpec((_N, 2), lambda: (0, 0)),
            pl.BlockSpec((_N, 2), lambda: (0, 0)),
        ],
        out_shape=[
            jax.ShapeDtypeStruct((_N, _E), jnp.float32),
            jax.ShapeDtypeStruct((_N, 2), jnp.int32),
            jax.ShapeDtypeStruct((_N, 2), jnp.float32),
        ],
    )(x, Wr)


def _sc_gather(x, gidx_flat):
    mesh = plsc.VectorSubcoreMesh(core_axis_name="c", subcore_axis_name="s")

    @functools.partial(
        pl.kernel,
        mesh=mesh,
        out_type=jax.ShapeDtypeStruct((2 * _N, _D), jnp.float32),
        scratch_types=[
            pltpu.VMEM((_CH,), jnp.int32),
            pltpu.VMEM((_CH, _D), jnp.float32),
            pltpu.SemaphoreType.DMA,
        ],
    )
    def k(x_hbm, gidx_hbm, xs_hbm, idx_v, rows_v, sem):
        wid = lax.axis_index("s") * _NC + lax.axis_index("c")
        base = wid * _RPW
        for c in range(_NCH):
            off = base + c * _CH
            pltpu.sync_copy(gidx_hbm.at[pl.ds(off, _CH)], idx_v)
            pltpu.async_copy(x_hbm.at[idx_v], rows_v, sem).wait()
            pltpu.sync_copy(rows_v, xs_hbm.at[pl.ds(off, _CH)])

    return k(x, gidx_flat)


_HC = 512
_NHT = _H // _HC


def _expert_body(xs_ref, w1x_ref, w1g_ref, w2_ref, gates_ref, out_ref):
    e = pl.program_id(0)
    ht = pl.program_id(1)
    h = pl.program_id(2)
    i = pl.program_id(3)
    x = xs_ref[0]
    hx = lax.dot_general(x, w1x_ref[0], (((1,), (1,)), ((), ())),
                         preferred_element_type=jnp.float32)
    hg = lax.dot_general(x, w1g_ref[0], (((1,), (1,)), ((), ())),
                         preferred_element_type=jnp.float32)
    act = (0.5 * hg * (1.0 + lax.erf(hg * 0.7071067811865476))) * hx
    y = lax.dot_general(act, w2_ref[0], (((1,), (1,)), ((), ())),
                        preferred_element_type=jnp.float32)
    rows0 = e * _TPE + i * _TT
    gcols = gates_ref[pl.ds(rows0, _TT), :]
    lane2 = lax.broadcasted_iota(jnp.int32, (_TT, 2), 1)
    g = jnp.sum(jnp.where(lane2 == h, gcols, 0.0), axis=-1, keepdims=True)
    contrib = y * g

    @pl.when((h == 0) & (ht == 0))
    def _init():
        out_ref[pl.ds(rows0, _TT), :] = contrib

    @pl.when((h == 1) | (ht != 0))
    def _acc():
        out_ref[pl.ds(rows0, _TT), :] += contrib


def _expert_call(xs, W1, W2, gates):
    return pl.pallas_call(
        _expert_body,
        grid=(_E, _NHT, 2, _NI),
        in_specs=[
            pl.BlockSpec((1, _TT, _D), lambda e, ht, h, i: (h, e * _NI + i, 0)),
            pl.BlockSpec((1, _HC, _D), lambda e, ht, h, i: (e, ht, 0)),
            pl.BlockSpec((1, _HC, _D), lambda e, ht, h, i: (e, _NHT + ht, 0)),
            pl.BlockSpec((1, _D, _HC), lambda e, ht, h, i: (e, 0, ht)),
            pl.BlockSpec((_N, 2), lambda e, ht, h, i: (0, 0)),
        ],
        out_specs=pl.BlockSpec((_N, _D), lambda e, ht, h, i: (0, 0)),
        out_shape=jax.ShapeDtypeStruct((_N, _D), jnp.float32),
    )(xs, W1, W1, W2, gates)


@jax.jit
def kernel(inputs, W1, W2, Wr):
    x = inputs.reshape(_N, _D)
    probs, gidx, gates = _router_call(x, Wr)
    gidx_flat = gidx.T.reshape(2 * _N)
    xs = _sc_gather(x, gidx_flat).reshape(2, _N, _D)
    out = _expert_call(xs, W1, W2, gates)
    return out.reshape(_B, _S, _D), probs.reshape(_B, _S, _E)
